# baseline (speedup 1.0000x reference)
import functools

import jax
import jax.numpy as jnp
from jax import lax
from jax.experimental import pallas as pl
from jax.experimental.pallas import tpu as pltpu
from jax.experimental.pallas import tpu_sc as plsc

D_MODEL = 2048
N_TOK = 16384
NC, NS, L = 2, 16, 16
NW = NC * NS
ROWS_PER_W = N_TOK // NW
CHUNK = 16
N_CHUNKS = ROWS_PER_W // CHUNK
BUF = CHUNK * D_MODEL


def _sc_body(zeros_hbm, idx_hbm, out_hbm, buf0, buf1, idx_v, sem0, sem1):
    bufs = (buf0, buf1)
    sems = (sem0, sem1)
    wid = lax.axis_index("s") * NC + lax.axis_index("c")
    base = wid * ROWS_PER_W
    pltpu.sync_copy(zeros_hbm, buf0)
    pltpu.sync_copy(zeros_hbm, buf1)
    pltpu.sync_copy(idx_hbm.at[pl.ds(base, ROWS_PER_W)], idx_v)
    rowoff = lax.iota(jnp.int32, L) * D_MODEL
    one = jnp.full((L,), 1.0, jnp.float32)
    zero = jnp.zeros((L,), jnp.float32)

    def out_slice(c):
        return out_hbm.at[pl.ds((base + c * CHUNK) * D_MODEL, BUF)]

    def step(g, _):
        for b in range(2):
            c = g * 2 + b

            @pl.when(c >= 2)
            def _drain():
                pltpu.make_async_copy(bufs[b], out_slice(c - 2), sems[b]).wait()
                cols_prev = idx_v[pl.ds((c - 2) * CHUNK, L)]
                plsc.store_scatter(bufs[b], [rowoff + cols_prev], zero)

            cols = idx_v[pl.ds(c * CHUNK, L)]
            plsc.store_scatter(bufs[b], [rowoff + cols], one)
            pltpu.make_async_copy(bufs[b], out_slice(c), sems[b]).start()
        return _

    lax.fori_loop(0, N_CHUNKS // 2, step, None)
    for b in range(2):
        c_last = N_CHUNKS - 2 + b
        pltpu.make_async_copy(bufs[b], out_slice(c_last), sems[b]).wait()


def kernel(x):
    b, s, _ = x.shape
    idx = x.reshape(N_TOK)
    zeros = jnp.zeros((BUF,), jnp.float32)
    mesh = plsc.VectorSubcoreMesh(core_axis_name="c", subcore_axis_name="s")
    k = functools.partial(
        pl.kernel,
        mesh=mesh,
        out_type=jax.ShapeDtypeStruct((N_TOK * D_MODEL,), jnp.float32),
        scratch_types=[
            pltpu.VMEM((BUF,), jnp.float32),
            pltpu.VMEM((BUF,), jnp.float32),
            pltpu.VMEM((ROWS_PER_W,), jnp.int32),
            pltpu.SemaphoreType.DMA,
            pltpu.SemaphoreType.DMA,
        ],
        compiler_params=pltpu.CompilerParams(needs_layout_passes=False),
    )(_sc_body)
    out = k(zeros, idx)
    return (out.reshape(b, s, D_MODEL),)


# speedup vs baseline: 2.8725x; 2.8725x over previous
import functools

import jax
import jax.numpy as jnp
from jax import lax
from jax.experimental import pallas as pl
from jax.experimental.pallas import tpu as pltpu
from jax.experimental.pallas import tpu_sc as plsc

D_MODEL = 2048
N_TOK = 16384
NC, NS, L = 2, 16, 16
NW = NC * NS
ROWS_PER_W = N_TOK // NW
CHUNK = 16
N_CHUNKS = ROWS_PER_W // CHUNK


def _sc_body(zeros_hbm, idx_hbm, out_hbm, buf0, buf1, idx_v, sem0, sem1):
    bufs = (buf0, buf1)
    sems = (sem0, sem1)
    wid = lax.axis_index("s") * NC + lax.axis_index("c")
    base = wid * ROWS_PER_W
    pltpu.sync_copy(zeros_hbm, buf0)
    pltpu.sync_copy(zeros_hbm, buf1)
    pltpu.sync_copy(idx_hbm.at[pl.ds(base, ROWS_PER_W)], idx_v)
    row16 = lax.iota(jnp.int32, L)
    one = jnp.full((L,), 1.0, jnp.float32)
    zero = jnp.zeros((L,), jnp.float32)

    def out_slice(c):
        return out_hbm.at[pl.ds(base + c * CHUNK, CHUNK)]

    def step(g, _):
        for b in range(2):
            c = g * 2 + b

            @pl.when(c >= 2)
            def _drain():
                pltpu.make_async_copy(bufs[b], out_slice(c - 2), sems[b]).wait()
                cols_prev = idx_v[pl.ds((c - 2) * CHUNK, L)]
                plsc.store_scatter(bufs[b], [row16, cols_prev], zero)

            cols = idx_v[pl.ds(c * CHUNK, L)]
            plsc.store_scatter(bufs[b], [row16, cols], one)
            pltpu.make_async_copy(bufs[b], out_slice(c), sems[b]).start()
        return _

    lax.fori_loop(0, N_CHUNKS // 2, step, None)
    for b in range(2):
        c_last = N_CHUNKS - 2 + b
        pltpu.make_async_copy(bufs[b], out_slice(c_last), sems[b]).wait()


def kernel(x):
    b, s, _ = x.shape
    idx = x.reshape(N_TOK)
    zeros = jnp.zeros((CHUNK, D_MODEL), jnp.float32)
    mesh = plsc.VectorSubcoreMesh(core_axis_name="c", subcore_axis_name="s")
    k = functools.partial(
        pl.kernel,
        mesh=mesh,
        out_type=jax.ShapeDtypeStruct((N_TOK, D_MODEL), jnp.float32),
        scratch_types=[
            pltpu.VMEM((CHUNK, D_MODEL), jnp.float32),
            pltpu.VMEM((CHUNK, D_MODEL), jnp.float32),
            pltpu.VMEM((ROWS_PER_W,), jnp.int32),
            pltpu.SemaphoreType.DMA,
            pltpu.SemaphoreType.DMA,
        ],
        compiler_params=pltpu.CompilerParams(needs_layout_passes=False),
    )(_sc_body)
    out = k(zeros, idx)
    return (out.reshape(b, s, D_MODEL),)
